# baseline (speedup 1.0000x reference)
import functools

import jax
import jax.numpy as jnp
from jax import lax
from jax.experimental import pallas as pl
from jax.experimental.pallas import tpu as pltpu
from jax.experimental.pallas import tpu_sc as plsc

NUM_F_MAPS = 64
B = 4096
L = 200

_NC, _NS = 2, 16
_NW = _NC * _NS
_CHUNK = 128
_TOTAL = B * L
_ROWS_PER_W = _TOTAL // _NW
_CHUNKS_PER_W = _ROWS_PER_W // _CHUNK


def _sc_gather(idx2d, table):
    mesh = plsc.VectorSubcoreMesh(core_axis_name="c", subcore_axis_name="s")

    @functools.partial(
        pl.kernel,
        mesh=mesh,
        compiler_params=pltpu.CompilerParams(use_tc_tiling_on_sc=False),
        out_type=[
            jax.ShapeDtypeStruct((_TOTAL, NUM_F_MAPS), jnp.float32),
            jax.ShapeDtypeStruct((_TOTAL // _CHUNK, _CHUNK), jnp.int32),
        ],
        scratch_types=[
            pltpu.VMEM((_CHUNKS_PER_W, _CHUNK), jnp.int32),
            pltpu.VMEM((_CHUNKS_PER_W, _CHUNK), jnp.int32),
            pltpu.VMEM((_CHUNK, NUM_F_MAPS), jnp.float32),
            pltpu.SemaphoreType.DMA,
        ],
    )
    def k(idx_hbm, table_hbm, emb_hbm, mask_hbm, idx_v, mask_v, rows_v, gsem):
        wid = lax.axis_index("s") * _NC + lax.axis_index("c")
        row0 = wid * _CHUNKS_PER_W

        pltpu.sync_copy(idx_hbm.at[pl.ds(row0, _CHUNKS_PER_W)], idx_v)

        def mask_body(r, _):
            for c in range(_CHUNK // 16):
                v = idx_v[r, pl.ds(c * 16, 16)]
                mask_v[r, pl.ds(c * 16, 16)] = jnp.where(v != -1, 1, 0)
            return _

        lax.fori_loop(0, _CHUNKS_PER_W, mask_body, None)
        pltpu.sync_copy(mask_v, mask_hbm.at[pl.ds(row0, _CHUNKS_PER_W)])

        def gather_body(j, _):
            pltpu.async_copy(table_hbm.at[idx_v.at[j]], rows_v, gsem).wait()
            pltpu.sync_copy(
                rows_v,
                emb_hbm.at[pl.ds(row0 * _CHUNK + j * _CHUNK, _CHUNK)],
            )
            return _

        lax.fori_loop(0, _CHUNKS_PER_W, gather_body, None)

    return k(idx2d, table)


def kernel(output, action_emb):
    labels = output[0].astype(jnp.int32).reshape(_TOTAL // _CHUNK, _CHUNK)
    emb_flat, mask2d = _sc_gather(labels, action_emb)
    tgt_emb = emb_flat.reshape(B, L, NUM_F_MAPS)
    mask_labels = mask2d.reshape(B, L)
    return (tgt_emb, mask_labels)


# speedup vs baseline: 1.1158x; 1.1158x over previous
import functools

import jax
import jax.numpy as jnp
from jax import lax
from jax.experimental import pallas as pl
from jax.experimental.pallas import tpu as pltpu
from jax.experimental.pallas import tpu_sc as plsc

NUM_F_MAPS = 64
B = 4096
L = 200

_NC, _NS = 2, 16
_NW = _NC * _NS
_CHUNK = 128
_K = 4
_TOTAL = B * L
_ROWS_PER_W = _TOTAL // _NW
_CHUNKS_PER_W = _ROWS_PER_W // _CHUNK
_GROUPS = _CHUNKS_PER_W // _K


def _sc_gather(idx2d, table):
    mesh = plsc.VectorSubcoreMesh(core_axis_name="c", subcore_axis_name="s")

    @functools.partial(
        pl.kernel,
        mesh=mesh,
        compiler_params=pltpu.CompilerParams(use_tc_tiling_on_sc=False),
        out_type=[
            jax.ShapeDtypeStruct((_TOTAL, NUM_F_MAPS), jnp.float32),
            jax.ShapeDtypeStruct((_TOTAL // _CHUNK, _CHUNK), jnp.int32),
        ],
        scratch_types=[
            pltpu.VMEM((_CHUNKS_PER_W, _CHUNK), jnp.int32),
            pltpu.VMEM((2, _K, _CHUNK, NUM_F_MAPS), jnp.float32),
            pltpu.SemaphoreType.DMA,
            pltpu.SemaphoreType.DMA,
            pltpu.SemaphoreType.DMA,
            pltpu.SemaphoreType.DMA,
        ],
    )
    def k(idx_hbm, table_hbm, emb_hbm, mask_hbm, idx_v, rows_v,
          gsem0, gsem1, wsem0, wsem1):
        wid = lax.axis_index("s") * _NC + lax.axis_index("c")
        row0 = wid * _CHUNKS_PER_W
        out0 = row0 * _CHUNK

        pltpu.sync_copy(idx_hbm.at[pl.ds(row0, _CHUNKS_PER_W)], idx_v)

        def gather(j, s, t, gsem):
            pltpu.async_copy(table_hbm.at[idx_v.at[j]], rows_v.at[s, t], gsem)

        def drain(sem, dst_is_hbm, s, t, j):
            if dst_is_hbm:
                pltpu.make_async_copy(
                    rows_v.at[s, t], emb_hbm.at[pl.ds(out0, _CHUNK)], sem
                ).wait()
            else:
                pltpu.make_async_copy(
                    table_hbm.at[idx_v.at[j]], rows_v.at[s, t], sem
                ).wait()

        def write(j, s, t, wsem):
            pltpu.async_copy(
                rows_v.at[s, t], emb_hbm.at[pl.ds(out0 + j * _CHUNK, _CHUNK)],
                wsem)

        def mask_row(j):
            for c in range(_CHUNK // 16):
                v = idx_v[j, pl.ds(c * 16, 16)]
                idx_v[j, pl.ds(c * 16, 16)] = jnp.where(v != -1, 1, 0)

        for t in range(_K):
            gather(t, 0, t, gsem0)

        def body(i, _):
            ga = 2 * i
            gb = 2 * i + 1

            @pl.when(i > 0)
            def _():
                for t in range(_K):
                    drain(wsem1, True, 1, t, 0)

            for t in range(_K):
                gather(gb * _K + t, 1, t, gsem1)

            for t in range(_K):
                drain(gsem0, False, 0, t, ga * _K + t)
            for t in range(_K):
                mask_row(ga * _K + t)
                write(ga * _K + t, 0, t, wsem0)
            for t in range(_K):
                drain(wsem0, True, 0, t, 0)

            @pl.when(i < _GROUPS // 2 - 1)
            def _():
                for t in range(_K):
                    gather((ga + 2) * _K + t, 0, t, gsem0)

            for t in range(_K):
                drain(gsem1, False, 1, t, gb * _K + t)
            for t in range(_K):
                mask_row(gb * _K + t)
                write(gb * _K + t, 1, t, wsem1)
            return _

        lax.fori_loop(0, _GROUPS // 2, body, None)

        for t in range(_K):
            drain(wsem1, True, 1, t, 0)

        pltpu.sync_copy(idx_v, mask_hbm.at[pl.ds(row0, _CHUNKS_PER_W)])

    return k(idx2d, table)


def kernel(output, action_emb):
    labels = output[0].astype(jnp.int32).reshape(_TOTAL // _CHUNK, _CHUNK)
    emb_flat, mask2d = _sc_gather(labels, action_emb)
    tgt_emb = emb_flat.reshape(B, L, NUM_F_MAPS)
    mask_labels = mask2d.reshape(B, L)
    return (tgt_emb, mask_labels)


# speedup vs baseline: 1.2605x; 1.1297x over previous
import functools

import jax
import jax.numpy as jnp
from jax import lax
from jax.experimental import pallas as pl
from jax.experimental.pallas import tpu as pltpu
from jax.experimental.pallas import tpu_sc as plsc

NUM_F_MAPS = 64
B = 4096
L = 200

_NC, _NS = 2, 16
_NW = _NC * _NS
_BPW = B // _NW
_K = 4
_GROUPS = L // _K


def _sc_gather(idx_t, table):
    mesh = plsc.VectorSubcoreMesh(core_axis_name="c", subcore_axis_name="s")

    @functools.partial(
        pl.kernel,
        mesh=mesh,
        compiler_params=pltpu.CompilerParams(use_tc_tiling_on_sc=False),
        out_type=[
            jax.ShapeDtypeStruct((B, L * NUM_F_MAPS), jnp.float32),
            jax.ShapeDtypeStruct((L, B), jnp.int32),
        ],
        scratch_types=[
            pltpu.VMEM((L, _BPW), jnp.int32),
            pltpu.VMEM((L, _BPW), jnp.int32),
            pltpu.VMEM((2, _K, _BPW, NUM_F_MAPS), jnp.float32),
            pltpu.SemaphoreType.DMA,
            pltpu.SemaphoreType.DMA,
            pltpu.SemaphoreType.DMA,
            pltpu.SemaphoreType.DMA,
        ],
    )
    def k(idx_hbm, table_hbm, emb_hbm, mask_hbm, idx_v, mask_v, rows_v,
          gsem0, gsem1, wsem0, wsem1):
        wid = lax.axis_index("s") * _NC + lax.axis_index("c")
        b0 = wid * _BPW

        pltpu.sync_copy(idx_hbm.at[0, :, pl.ds(b0, _BPW)], idx_v)

        def gather(ell, s, t, gsem):
            pltpu.async_copy(table_hbm.at[idx_v.at[ell]], rows_v.at[s, t],
                             gsem)

        def write(ell, s, t, wsem):
            pltpu.async_copy(
                rows_v.at[s, t],
                emb_hbm.at[pl.ds(b0, _BPW), pl.ds(ell * NUM_F_MAPS,
                                                  NUM_F_MAPS)],
                wsem)

        def drain_gather(s, t, sem):
            pltpu.make_async_copy(
                table_hbm.at[idx_v.at[0]], rows_v.at[s, t], sem).wait()

        def drain_write(s, t, sem):
            pltpu.make_async_copy(
                rows_v.at[s, t],
                emb_hbm.at[pl.ds(b0, _BPW), pl.ds(0, NUM_F_MAPS)],
                sem).wait()

        def mask_row(ell):
            for c in range(_BPW // 16):
                v = idx_v[ell, pl.ds(c * 16, 16)]
                mask_v[ell, pl.ds(c * 16, 16)] = jnp.where(v != -1, 1, 0)

        for t in range(_K):
            gather(t, 0, t, gsem0)

        def body(i, _):
            ga = 2 * i
            gb = 2 * i + 1

            @pl.when(i > 0)
            def _():
                for t in range(_K):
                    drain_write(1, t, wsem1)

            for t in range(_K):
                gather(gb * _K + t, 1, t, gsem1)

            for t in range(_K):
                drain_gather(0, t, gsem0)
            for t in range(_K):
                mask_row(ga * _K + t)
                write(ga * _K + t, 0, t, wsem0)
            for t in range(_K):
                drain_write(0, t, wsem0)

            @pl.when(i < _GROUPS // 2 - 1)
            def _():
                for t in range(_K):
                    gather((ga + 2) * _K + t, 0, t, gsem0)

            for t in range(_K):
                drain_gather(1, t, gsem1)
            for t in range(_K):
                mask_row(gb * _K + t)
                write(gb * _K + t, 1, t, wsem1)
            return _

        lax.fori_loop(0, _GROUPS // 2, body, None)

        for t in range(_K):
            drain_write(1, t, wsem1)

        pltpu.sync_copy(mask_v, mask_hbm.at[:, pl.ds(b0, _BPW)])

    return k(idx_t, table)


def kernel(output, action_emb):
    idx_t = jnp.transpose(output, (0, 2, 1))
    emb2d, mask_t = _sc_gather(idx_t, action_emb)
    tgt_emb = emb2d.reshape(B, L, NUM_F_MAPS)
    mask_labels = mask_t.T
    return (tgt_emb, mask_labels)
